# Initial kernel scaffold; baseline (speedup 1.0000x reference)
import jax
import jax.numpy as jnp
from jax.experimental import pallas as pl


def kernel(coordinates, row_splits, bin_width, nbins, return_all):
    raise NotImplementedError("write your pallas kernel here")


# speedup vs baseline: 26.3433x; 26.3433x over previous
import jax
import jax.numpy as jnp
from jax import lax
from jax.experimental import pallas as pl
from jax.experimental.pallas import tpu as pltpu
from jax.experimental.pallas import tpu_sc as plsc

_NC = 2
_NS = 16
_L = 16
_G = 125
_CP = _G * _L


def _i32(x):
    return jnp.asarray(x, dtype=jnp.int32)


def _make_sc_kernel(N, D, n_rows, HB):
    NCH = N // _CP
    NW = _NC * _NS
    K1 = -(-NCH // _NS)
    K2 = -(-NCH // NW)
    HSL = HB // _NS
    CW = _CP * D
    NB_OFF = (n_rows - 1) * _L

    def body(coords, pf, pi, binass_o, flat_o, hist_o,
             cbuf, fidx, bbuf, onesv, pfv, piv, aminv, minv, zv,
             shist, smin):
        c = lax.axis_index("c")
        s = lax.axis_index("s")
        wid = s * _NC + c
        lanes = lax.iota(jnp.int32, _L)
        idx3 = lanes * 3
        idx4 = lanes * 4
        zero16 = jnp.zeros((_L,), jnp.int32)
        one16 = jnp.full((_L,), 1, jnp.int32)
        inf16 = jnp.full((_L,), jnp.inf, jnp.float32)

        pltpu.sync_copy(pf, pfv)
        pltpu.sync_copy(pi, piv)

        def _fill_ones(i, _):
            onesv[pl.ds(i * _L, _L)] = one16
            return 0
        lax.fori_loop(_i32(0), _i32(_G), _fill_ones, 0)

        def _fill_z(i, _):
            zv[pl.ds(i * _L, _L)] = zero16
            return 0
        lax.fori_loop(_i32(0), _i32(HSL // _L), _fill_z, 0)
        pltpu.sync_copy(zv, shist.at[pl.ds(s * HSL, HSL)])

        def _min_chunk(k, carry):
            ch = jnp.minimum(s + _NS * k, jnp.int32(NCH - 1))
            pltpu.sync_copy(coords.at[pl.ds(ch * CW, CW)], cbuf)

            def _g(g, mm):
                a0, a1, a2 = mm
                b = g * 48
                return (jnp.minimum(a0, cbuf[pl.ds(b, _L)]),
                        jnp.minimum(a1, cbuf[pl.ds(b + 16, _L)]),
                        jnp.minimum(a2, cbuf[pl.ds(b + 32, _L)]))
            return lax.fori_loop(_i32(0), _i32(_G), _g, carry)
        m0, m1, m2 = lax.fori_loop(_i32(0), _i32(K1), _min_chunk, (inf16, inf16, inf16))
        minv[pl.ds(0, _L)] = m0
        minv[pl.ds(16, _L)] = m1
        minv[pl.ds(32, _L)] = m2
        pltpu.sync_copy(minv, smin.at[pl.ds(s * 48, 48)])
        plsc.subcore_barrier()
        pltpu.sync_copy(smin, aminv)

        def _red(t, mm):
            a0, a1, a2 = mm
            b = t * 48
            return (jnp.minimum(a0, aminv[pl.ds(b, _L)]),
                    jnp.minimum(a1, aminv[pl.ds(b + 16, _L)]),
                    jnp.minimum(a2, aminv[pl.ds(b + 32, _L)]))
        gm = lax.fori_loop(_i32(0), _i32(_NS), _red, (inf16, inf16, inf16))
        dmin = []
        for d in range(D):
            md = inf16
            for k in range(3):
                msk = ((lanes + k) % 3) == d
                md = jnp.minimum(md, jnp.where(msk, gm[k], inf16))
            dmin.append(jnp.broadcast_to(jnp.min(md), (_L,)))

        bw = pfv[pl.ds(0, _L)]
        nf = [pfv[pl.ds((1 + d) * _L, _L)] for d in range(D)]
        rsv = [piv[pl.ds(r * _L, _L)] for r in range(n_rows - 1)]
        nb = [piv[pl.ds(NB_OFF + d * _L, _L)] for d in range(D)]

        def _chunk2(k, _):
            ch = wid + NW * k

            @pl.when(ch < NCH)
            def _():
                pltpu.sync_copy(coords.at[pl.ds(ch * CW, CW)], cbuf)
                ptb0 = ch * _CP

                def _g(g, _c):
                    b = g * 48
                    xx = plsc.load_gather(cbuf, [b + idx3])
                    yy = plsc.load_gather(cbuf, [b + idx3 + 1])
                    zz = plsc.load_gather(cbuf, [b + idx3 + 2])
                    ii = []
                    for d, vv in enumerate((xx, yy, zz)):
                        q = (vv - dmin[d]) / bw
                        q = jnp.minimum(q, nf[d])
                        ii.append(q.astype(jnp.int32))
                    pt = ptb0 + g * _L + lanes
                    r = zero16
                    for rb in rsv:
                        r = r + jnp.where(pt >= rb, one16, zero16)
                    fl = r
                    for d in range(D):
                        fl = fl * nb[d] + ii[d]
                    fidx[pl.ds(g * _L, _L)] = fl
                    b4 = g * (4 * _L)
                    plsc.store_scatter(bbuf, [b4 + idx4], r)
                    plsc.store_scatter(bbuf, [b4 + idx4 + 1], ii[0])
                    plsc.store_scatter(bbuf, [b4 + idx4 + 2], ii[1])
                    plsc.store_scatter(bbuf, [b4 + idx4 + 3], ii[2])
                    return 0
                lax.fori_loop(_i32(0), _i32(_G), _g, 0)
                pltpu.sync_copy(fidx, flat_o.at[pl.ds(ch * _CP, _CP)])
                pltpu.sync_copy(bbuf, binass_o.at[pl.ds(ch * 4 * _CP, 4 * _CP)])
                pltpu.sync_copy(onesv, shist.at[fidx], add=True)
            return 0
        lax.fori_loop(_i32(0), _i32(K2), _chunk2, 0)

        plsc.subcore_barrier()
        pltpu.sync_copy(shist.at[pl.ds(s * HSL, HSL)], zv)
        pltpu.sync_copy(zv, hist_o.at[pl.ds(c * HB + s * HSL, HSL)])

    mesh = plsc.VectorSubcoreMesh(core_axis_name="c", subcore_axis_name="s")
    return pl.kernel(
        body,
        out_type=(
            jax.ShapeDtypeStruct((N * 4,), jnp.int32),
            jax.ShapeDtypeStruct((N,), jnp.int32),
            jax.ShapeDtypeStruct((_NC * HB,), jnp.int32),
        ),
        mesh=mesh,
        compiler_params=pltpu.CompilerParams(needs_layout_passes=False),
        scratch_types=[
            pltpu.VMEM((CW,), jnp.float32),
            pltpu.VMEM((_CP,), jnp.int32),
            pltpu.VMEM((4 * _CP,), jnp.int32),
            pltpu.VMEM((_CP,), jnp.int32),
            pltpu.VMEM(((1 + D) * _L,), jnp.float32),
            pltpu.VMEM(((n_rows - 1 + D) * _L,), jnp.int32),
            pltpu.VMEM((_NS * 48,), jnp.float32),
            pltpu.VMEM((48,), jnp.float32),
            pltpu.VMEM((HSL,), jnp.int32),
            pltpu.VMEM_SHARED((HB,), jnp.int32),
            pltpu.VMEM_SHARED((_NS * 48,), jnp.float32),
        ],
    )


def _tc_add(hist2, HB):
    def bdy(h_ref, o_ref):
        o_ref[...] = h_ref[0] + h_ref[1]
    out = pl.pallas_call(
        bdy,
        out_shape=jax.ShapeDtypeStruct((HB // 128, 128), jnp.int32),
    )(hist2.reshape(_NC, HB // 128, 128))
    return out.reshape(HB)


def kernel(coordinates, row_splits, bin_width, nbins, return_all):
    N, D = coordinates.shape
    n_rows = row_splits.shape[0] - 1
    HB = n_rows * 24 ** D

    coords_flat = coordinates.reshape(N * D)
    rs32 = row_splits.astype(jnp.int32)
    pi = jnp.concatenate([
        jnp.broadcast_to(rs32[1:n_rows, None], (n_rows - 1, _L)),
        jnp.broadcast_to(nbins[:, None], (D, _L)),
    ], axis=0).reshape(-1)
    pf = jnp.concatenate([
        jnp.broadcast_to(bin_width[:, None], (1, _L)),
        jnp.broadcast_to((nbins[:, None] - 1).astype(jnp.float32), (D, _L)),
    ], axis=0).reshape(-1)

    fn = _make_sc_kernel(N, D, n_rows, HB)
    binass2, flat2, hist2 = fn(coords_flat, pf, pi)
    binass = binass2.reshape(N, D + 1)
    flat = flat2.reshape(N)
    n_per_bin = _tc_add(hist2, HB)
    return binass, flat, nbins, bin_width, n_per_bin


# speedup vs baseline: 446.7846x; 16.9601x over previous
import jax
import jax.numpy as jnp
from jax import lax
from jax.experimental import pallas as pl
from jax.experimental.pallas import tpu as pltpu
from jax.experimental.pallas import tpu_sc as plsc

_NC = 2
_NS = 16
_L = 16
_BP = 128
_CB = 16
_CPTS = _CB * _BP
_CG = _CPTS // _L


def _i32(x):
    return jnp.asarray(x, dtype=jnp.int32)


def _make_sc_kernel(N, D, n_rows, HB):
    NW = _NC * _NS
    NCHF = N // _CPTS
    TP = N - NCHF * _CPTS
    NCH = NCHF + (1 if TP else 0)
    TG = TP // _L
    NBLK = -(-N // _BP)
    TBW = (NBLK - NCHF * _CB) * 4 * _BP
    OUTW = NBLK * 4 * _BP
    K1 = -(-NCHF // _NS)
    K2 = -(-NCH // NW)
    HSL = HB // _NS
    NB_OFF = (n_rows - 1) * _L

    def body(coords, pf, pi, binass_o, flat_o, hist_o,
             xbuf, ybuf, zbuf, fidx, bbuf, onesv, fidx_t, ones_t,
             pfv, piv, aminv, minv, zv, shist, smin):
        c = lax.axis_index("c")
        s = lax.axis_index("s")
        wid = s * _NC + c
        lanes = lax.iota(jnp.int32, _L)
        zero16 = jnp.zeros((_L,), jnp.int32)
        one16 = jnp.full((_L,), 1, jnp.int32)
        inf16 = jnp.full((_L,), jnp.inf, jnp.float32)
        bufs = (xbuf, ybuf, zbuf)

        pltpu.sync_copy(pf, pfv)
        pltpu.sync_copy(pi, piv)

        def _fill_ones(i, _):
            onesv[pl.ds(i * _L, _L)] = one16
            return 0
        lax.fori_loop(_i32(0), _i32(_CG), _fill_ones, 0)

        def _fill_ones_t(i, _):
            ones_t[pl.ds(i * _L, _L)] = one16
            return 0
        lax.fori_loop(_i32(0), _i32(TG), _fill_ones_t, 0)

        def _fill_z(i, _):
            zv[pl.ds(i * _L, _L)] = zero16
            return 0
        lax.fori_loop(_i32(0), _i32(HSL // _L), _fill_z, 0)
        pltpu.sync_copy(zv, shist.at[pl.ds(s * HSL, HSL)])

        def _min_chunk(k, carry):
            ch = jnp.minimum(s + _NS * k, _i32(NCHF - 1))
            for d in range(D):
                pltpu.sync_copy(coords.at[pl.ds(d * N + ch * _CPTS, _CPTS)],
                                bufs[d])

            def _g(g, mm):
                b = g * _L
                return tuple(jnp.minimum(mm[d], bufs[d][pl.ds(b, _L)])
                             for d in range(D))
            return lax.fori_loop(_i32(0), _i32(_CG), _g, carry)
        mins = lax.fori_loop(_i32(0), _i32(K1), _min_chunk, (inf16,) * D)
        if TP:
            for d in range(D):
                pltpu.sync_copy(coords.at[pl.ds(d * N + NCHF * _CPTS, TP)],
                                bufs[d].at[pl.ds(0, TP)])

            def _gt(g, mm):
                b = g * _L
                return tuple(jnp.minimum(mm[d], bufs[d][pl.ds(b, _L)])
                             for d in range(D))
            mins = lax.fori_loop(_i32(0), _i32(TG), _gt, mins)
        for d in range(D):
            minv[pl.ds(d * _L, _L)] = mins[d]
        pltpu.sync_copy(minv, smin.at[pl.ds(s * D * _L, D * _L)])
        plsc.subcore_barrier()
        pltpu.sync_copy(smin, aminv)

        def _red(t, mm):
            b = t * (D * _L)
            return tuple(jnp.minimum(mm[d], aminv[pl.ds(b + d * _L, _L)])
                         for d in range(D))
        gm = lax.fori_loop(_i32(0), _i32(_NS), _red, (inf16,) * D)
        dmin = [jnp.broadcast_to(jnp.min(gm[d]), (_L,)) for d in range(D)]

        bw = pfv[pl.ds(0, _L)]
        nf = [pfv[pl.ds((1 + d) * _L, _L)] for d in range(D)]
        rsv = [piv[pl.ds(r * _L, _L)] for r in range(n_rows - 1)]
        nb = [piv[pl.ds(NB_OFF + d * _L, _L)] for d in range(D)]

        def _group_body(ptb0, fb, g):
            b = g * _L
            ii = []
            for d in range(D):
                q = (bufs[d][pl.ds(b, _L)] - dmin[d]) / bw
                q = jnp.minimum(q, nf[d])
                ii.append(q.astype(jnp.int32))
            pt = ptb0 + g * _L + lanes
            r = zero16
            for rb in rsv:
                r = r + jnp.where(pt >= rb, one16, zero16)
            fl = r
            for d in range(D):
                fl = fl * nb[d] + ii[d]
            fb[pl.ds(b, _L)] = fl
            base = (g // 8) * (4 * _BP) + (g % 8) * _L
            bbuf[pl.ds(base, _L)] = r
            for d in range(D):
                bbuf[pl.ds(base + (d + 1) * _BP, _L)] = ii[d]

        def _chunk2(k, _):
            ch = wid + NW * k

            @pl.when(ch < NCHF)
            def _():
                for d in range(D):
                    pltpu.sync_copy(coords.at[pl.ds(d * N + ch * _CPTS, _CPTS)],
                                    bufs[d])
                ptb0 = ch * _CPTS

                def _g(g, _c):
                    _group_body(ptb0, fidx, g)
                    return 0
                lax.fori_loop(_i32(0), _i32(_CG), _g, 0)
                pltpu.sync_copy(fidx, flat_o.at[pl.ds(ch * _CPTS, _CPTS)])
                pltpu.sync_copy(bbuf, binass_o.at[pl.ds(ch * 4 * _CPTS,
                                                        4 * _CPTS)])
                pltpu.sync_copy(onesv, shist.at[fidx], add=True)

            if TP:
                @pl.when(ch == NCHF)
                def _():
                    for d in range(D):
                        pltpu.sync_copy(
                            coords.at[pl.ds(d * N + NCHF * _CPTS, TP)],
                            bufs[d].at[pl.ds(0, TP)])
                    ptb0 = _i32(NCHF * _CPTS)

                    def _g(g, _c):
                        _group_body(ptb0, fidx_t, g)
                        return 0
                    lax.fori_loop(_i32(0), _i32(TG), _g, 0)
                    pltpu.sync_copy(fidx_t, flat_o.at[pl.ds(NCHF * _CPTS, TP)])
                    pltpu.sync_copy(bbuf.at[pl.ds(0, TBW)],
                                    binass_o.at[pl.ds(NCHF * 4 * _CPTS, TBW)])
                    pltpu.sync_copy(ones_t, shist.at[fidx_t], add=True)
            return 0
        lax.fori_loop(_i32(0), _i32(K2), _chunk2, 0)

        plsc.subcore_barrier()
        pltpu.sync_copy(shist.at[pl.ds(s * HSL, HSL)], zv)
        pltpu.sync_copy(zv, hist_o.at[pl.ds(c * HB + s * HSL, HSL)])

    mesh = plsc.VectorSubcoreMesh(core_axis_name="c", subcore_axis_name="s")
    return pl.kernel(
        body,
        out_type=(
            jax.ShapeDtypeStruct((OUTW,), jnp.int32),
            jax.ShapeDtypeStruct((N,), jnp.int32),
            jax.ShapeDtypeStruct((_NC * HB,), jnp.int32),
        ),
        mesh=mesh,
        compiler_params=pltpu.CompilerParams(needs_layout_passes=False),
        scratch_types=[
            pltpu.VMEM((_CPTS,), jnp.float32),
            pltpu.VMEM((_CPTS,), jnp.float32),
            pltpu.VMEM((_CPTS,), jnp.float32),
            pltpu.VMEM((_CPTS,), jnp.int32),
            pltpu.VMEM((4 * _CPTS,), jnp.int32),
            pltpu.VMEM((_CPTS,), jnp.int32),
            pltpu.VMEM((max(TP, _L),), jnp.int32),
            pltpu.VMEM((max(TP, _L),), jnp.int32),
            pltpu.VMEM(((1 + D) * _L,), jnp.float32),
            pltpu.VMEM(((n_rows - 1 + D) * _L,), jnp.int32),
            pltpu.VMEM((_NS * 3 * _L,), jnp.float32),
            pltpu.VMEM((3 * _L,), jnp.float32),
            pltpu.VMEM((HSL,), jnp.int32),
            pltpu.VMEM_SHARED((HB,), jnp.int32),
            pltpu.VMEM_SHARED((_NS * 3 * _L,), jnp.float32),
        ],
    )


def _tc_add(hist2, HB):
    def bdy(h_ref, o_ref):
        o_ref[...] = h_ref[0] + h_ref[1]
    out = pl.pallas_call(
        bdy,
        out_shape=jax.ShapeDtypeStruct((HB // 128, 128), jnp.int32),
    )(hist2.reshape(_NC, HB // 128, 128))
    return out.reshape(HB)


def kernel(coordinates, row_splits, bin_width, nbins, return_all):
    N, D = coordinates.shape
    n_rows = row_splits.shape[0] - 1
    HB = n_rows * 24 ** D
    NBLK = -(-N // _BP)

    coords_planes = coordinates.T.reshape(N * D)
    rs32 = row_splits.astype(jnp.int32)
    pi = jnp.concatenate([
        jnp.broadcast_to(rs32[1:n_rows, None], (n_rows - 1, _L)),
        jnp.broadcast_to(nbins[:, None], (D, _L)),
    ], axis=0).reshape(-1)
    pf = jnp.concatenate([
        jnp.broadcast_to(bin_width[:, None], (1, _L)),
        jnp.broadcast_to((nbins[:, None] - 1).astype(jnp.float32), (D, _L)),
    ], axis=0).reshape(-1)

    fn = _make_sc_kernel(N, D, n_rows, HB)
    binass_blocks, flat, hist2 = fn(coords_planes, pf, pi)
    binass = (binass_blocks.reshape(NBLK, D + 1, _BP)
              .transpose(0, 2, 1).reshape(NBLK * _BP, D + 1)[:N])
    n_per_bin = _tc_add(hist2, HB)
    return binass, flat, nbins, bin_width, n_per_bin


# speedup vs baseline: 542.5652x; 1.2144x over previous
import jax
import jax.numpy as jnp
from jax import lax
from jax.experimental import pallas as pl
from jax.experimental.pallas import tpu as pltpu
from jax.experimental.pallas import tpu_sc as plsc

_NC = 2
_NS = 16
_L = 16
_BP = 128
_CB = 32
_CPTS = _CB * _BP
_CG = _CPTS // _L
_UG = 4
_UM = 8


def _i32(x):
    return jnp.asarray(x, dtype=jnp.int32)


def _make_sc_kernel(N, D, n_rows, HB):
    NW = _NC * _NS
    NCHF = N // _CPTS
    TP = N - NCHF * _CPTS
    NCH = NCHF + (1 if TP else 0)
    TG = TP // _L
    NBLK = -(-N // _BP)
    TBW = (NBLK - NCHF * _CB) * 4 * _BP
    OUTW = NBLK * 4 * _BP
    K1 = -(-NCHF // _NS)
    K2 = -(-NCH // NW)
    HSL = HB // _NS
    NB_OFF = (n_rows - 1) * _L

    def body(coords, pf, pi, binass_o, flat_o, hist_o,
             xbuf, ybuf, zbuf, fidx, bbuf, onesv, fidx_t, ones_t,
             pfv, piv, aminv, minv, zv, shist, smin):
        c = lax.axis_index("c")
        s = lax.axis_index("s")
        wid = s * _NC + c
        lanes = lax.iota(jnp.int32, _L)
        zero16 = jnp.zeros((_L,), jnp.int32)
        one16 = jnp.full((_L,), 1, jnp.int32)
        inf16 = jnp.full((_L,), jnp.inf, jnp.float32)
        bufs = (xbuf, ybuf, zbuf)

        pltpu.sync_copy(pf, pfv)
        pltpu.sync_copy(pi, piv)

        def _fill_ones(i, _):
            onesv[pl.ds(i * _L, _L)] = one16
            return 0
        lax.fori_loop(_i32(0), _i32(_CG), _fill_ones, 0)

        def _fill_ones_t(i, _):
            ones_t[pl.ds(i * _L, _L)] = one16
            return 0
        lax.fori_loop(_i32(0), _i32(TG), _fill_ones_t, 0)

        def _fill_z(i, _):
            zv[pl.ds(i * _L, _L)] = zero16
            return 0
        lax.fori_loop(_i32(0), _i32(HSL // _L), _fill_z, 0)
        pltpu.sync_copy(zv, shist.at[pl.ds(s * HSL, HSL)])

        def _min_chunk(k, carry):
            ch = jnp.minimum(s + _NS * k, _i32(NCHF - 1))
            for d in range(D):
                pltpu.sync_copy(coords.at[pl.ds(d * N + ch * _CPTS, _CPTS)],
                                bufs[d])

            def _g(g, mm):
                b = g * (_L * _UM)
                for u in range(_UM):
                    mm = tuple(
                        jnp.minimum(mm[d],
                                    bufs[d][pl.ds(b + u * _L, _L)])
                        for d in range(D))
                return mm
            return lax.fori_loop(_i32(0), _i32(_CG // _UM), _g, carry)
        mins = lax.fori_loop(_i32(0), _i32(K1), _min_chunk, (inf16,) * D)
        if TP:
            for d in range(D):
                pltpu.sync_copy(coords.at[pl.ds(d * N + NCHF * _CPTS, TP)],
                                bufs[d].at[pl.ds(0, TP)])

            def _gt(g, mm):
                b = g * _L
                return tuple(jnp.minimum(mm[d], bufs[d][pl.ds(b, _L)])
                             for d in range(D))
            mins = lax.fori_loop(_i32(0), _i32(TG), _gt, mins)
        for d in range(D):
            minv[pl.ds(d * _L, _L)] = mins[d]
        pltpu.sync_copy(minv, smin.at[pl.ds(s * D * _L, D * _L)])
        plsc.subcore_barrier()
        pltpu.sync_copy(smin, aminv)

        def _red(t, mm):
            b = t * (D * _L)
            return tuple(jnp.minimum(mm[d], aminv[pl.ds(b + d * _L, _L)])
                         for d in range(D))
        gm = lax.fori_loop(_i32(0), _i32(_NS), _red, (inf16,) * D)
        dmin = [jnp.broadcast_to(jnp.min(gm[d]), (_L,)) for d in range(D)]

        bw = pfv[pl.ds(0, _L)]
        invbw = jnp.full((_L,), 1.0, jnp.float32) / bw
        nf = [pfv[pl.ds((1 + d) * _L, _L)] for d in range(D)]
        rsv = [piv[pl.ds(r * _L, _L)] for r in range(n_rows - 1)]
        nb = [piv[pl.ds(NB_OFF + d * _L, _L)] for d in range(D)]

        def _rowid(pt):
            r = zero16
            for rb in rsv:
                r = r + jnp.where(pt >= rb, one16, zero16)
            return r

        def _one_group(ptb0, fb, g, u, unroll, r_const):
            b = g * (_L * unroll) + u * _L
            ii = []
            for d in range(D):
                q = (bufs[d][pl.ds(b, _L)] - dmin[d]) * invbw
                q = jnp.minimum(q, nf[d])
                ii.append(q.astype(jnp.int32))
            if r_const is None:
                r = _rowid(ptb0 + b + lanes)
            else:
                r = r_const
            fl = r
            for d in range(D):
                fl = fl * nb[d] + ii[d]
            fb[pl.ds(b, _L)] = fl
            gg = b // _L
            base = (gg // 8) * (4 * _BP) + (gg % 8) * _L
            bbuf[pl.ds(base, _L)] = r
            for d in range(D):
                bbuf[pl.ds(base + (d + 1) * _BP, _L)] = ii[d]

        def _run_groups(ptb0, fb, nsteps, unroll, r_const):
            def _g(g, _c):
                for u in range(unroll):
                    _one_group(ptb0, fb, g, u, unroll, r_const)
                return 0
            lax.fori_loop(_i32(0), _i32(nsteps), _g, 0)

        def _chunk2(k, _):
            ch = wid + NW * k

            @pl.when(ch < NCHF)
            def _():
                for d in range(D):
                    pltpu.sync_copy(coords.at[pl.ds(d * N + ch * _CPTS, _CPTS)],
                                    bufs[d])
                ptb0 = ch * _CPTS
                _run_groups(ptb0, fidx, _CG // _UG, _UG, None)
                pltpu.sync_copy(fidx, flat_o.at[pl.ds(ch * _CPTS, _CPTS)])
                pltpu.sync_copy(bbuf, binass_o.at[pl.ds(ch * 4 * _CPTS,
                                                        4 * _CPTS)])
                pltpu.sync_copy(onesv, shist.at[fidx], add=True)

            if TP:
                @pl.when(ch == NCHF)
                def _():
                    for d in range(D):
                        pltpu.sync_copy(
                            coords.at[pl.ds(d * N + NCHF * _CPTS, TP)],
                            bufs[d].at[pl.ds(0, TP)])
                    ptb0 = _i32(NCHF * _CPTS)
                    _run_groups(ptb0, fidx_t, TG, 1, None)
                    pltpu.sync_copy(fidx_t, flat_o.at[pl.ds(NCHF * _CPTS, TP)])
                    pltpu.sync_copy(bbuf.at[pl.ds(0, TBW)],
                                    binass_o.at[pl.ds(NCHF * 4 * _CPTS, TBW)])
                    pltpu.sync_copy(ones_t, shist.at[fidx_t], add=True)
            return 0
        lax.fori_loop(_i32(0), _i32(K2), _chunk2, 0)

        plsc.subcore_barrier()
        pltpu.sync_copy(shist.at[pl.ds(s * HSL, HSL)], zv)
        pltpu.sync_copy(zv, hist_o.at[pl.ds(c * HB + s * HSL, HSL)])

    mesh = plsc.VectorSubcoreMesh(core_axis_name="c", subcore_axis_name="s")
    return pl.kernel(
        body,
        out_type=(
            jax.ShapeDtypeStruct((OUTW,), jnp.int32),
            jax.ShapeDtypeStruct((N,), jnp.int32),
            jax.ShapeDtypeStruct((_NC * HB,), jnp.int32),
        ),
        mesh=mesh,
        compiler_params=pltpu.CompilerParams(needs_layout_passes=False),
        scratch_types=[
            pltpu.VMEM((_CPTS,), jnp.float32),
            pltpu.VMEM((_CPTS,), jnp.float32),
            pltpu.VMEM((_CPTS,), jnp.float32),
            pltpu.VMEM((_CPTS,), jnp.int32),
            pltpu.VMEM((4 * _CPTS,), jnp.int32),
            pltpu.VMEM((_CPTS,), jnp.int32),
            pltpu.VMEM((max(TP, _L),), jnp.int32),
            pltpu.VMEM((max(TP, _L),), jnp.int32),
            pltpu.VMEM(((1 + D) * _L,), jnp.float32),
            pltpu.VMEM(((n_rows - 1 + D) * _L,), jnp.int32),
            pltpu.VMEM((_NS * 3 * _L,), jnp.float32),
            pltpu.VMEM((3 * _L,), jnp.float32),
            pltpu.VMEM((HSL,), jnp.int32),
            pltpu.VMEM_SHARED((HB,), jnp.int32),
            pltpu.VMEM_SHARED((_NS * 3 * _L,), jnp.float32),
        ],
    )


def _tc_add(hist2, HB):
    def bdy(h_ref, o_ref):
        o_ref[...] = h_ref[0] + h_ref[1]
    out = pl.pallas_call(
        bdy,
        out_shape=jax.ShapeDtypeStruct((HB // 128, 128), jnp.int32),
    )(hist2.reshape(_NC, HB // 128, 128))
    return out.reshape(HB)


def kernel(coordinates, row_splits, bin_width, nbins, return_all):
    N, D = coordinates.shape
    n_rows = row_splits.shape[0] - 1
    HB = n_rows * 24 ** D
    NBLK = -(-N // _BP)

    coords_planes = coordinates.T.reshape(N * D)
    rs32 = row_splits.astype(jnp.int32)
    pi = jnp.concatenate([
        jnp.broadcast_to(rs32[1:n_rows, None], (n_rows - 1, _L)),
        jnp.broadcast_to(nbins[:, None], (D, _L)),
    ], axis=0).reshape(-1)
    pf = jnp.concatenate([
        jnp.broadcast_to(bin_width[:, None], (1, _L)),
        jnp.broadcast_to((nbins[:, None] - 1).astype(jnp.float32), (D, _L)),
    ], axis=0).reshape(-1)

    fn = _make_sc_kernel(N, D, n_rows, HB)
    binass_blocks, flat, hist2 = fn(coords_planes, pf, pi)
    binass = (binass_blocks.reshape(NBLK, D + 1, _BP)
              .transpose(0, 2, 1).reshape(NBLK * _BP, D + 1)[:N])
    n_per_bin = _tc_add(hist2, HB)
    return binass, flat, nbins, bin_width, n_per_bin
